# Initial kernel scaffold; baseline (speedup 1.0000x reference)
import functools

import jax
import jax.numpy as jnp
from jax import lax
from jax.experimental import pallas as pl
from jax.experimental.pallas import tpu as pltpu
from jax.experimental.pallas import tpu_sc as plsc

B, N, T = 1024, 26, 20
M = B * N * T
D = 16
NC, NS = 2, 16
NW = NC * NS
CHUNK = M // NW
SUB = 2080
N_ITERS = CHUNK // SUB

_mesh = plsc.VectorSubcoreMesh(core_axis_name="c", subcore_axis_name="s")


@functools.partial(
    pl.kernel,
    mesh=_mesh,
    out_type=jax.ShapeDtypeStruct((M, 3 * D), jnp.float32),
    scratch_types=[
        pltpu.VMEM((SUB,), jnp.int32),
        pltpu.VMEM((SUB, D), jnp.float32),
        pltpu.SemaphoreType.DMA,
    ],
)
def _embed(idx0, idx1, idx2, w0, w1, w2, out, idx_v, rows_v, sem):
    wid = lax.axis_index("s") * NC + lax.axis_index("c")
    base = wid * CHUNK
    idxs = (idx0, idx1, idx2)
    tables = (w0, w1, w2)

    def body(j, _):
        start = base + j * SUB
        for i in range(3):
            pltpu.sync_copy(idxs[i].at[pl.ds(start, SUB)], idx_v)
            pltpu.async_copy(tables[i].at[idx_v], rows_v, sem).wait()
            pltpu.sync_copy(rows_v, out.at[pl.ds(start, SUB), pl.ds(i * D, D)])
        return 0

    lax.fori_loop(0, N_ITERS, body, 0)


def kernel(x, W0, W1, W2):
    xf = x.reshape(M, 3)
    out = _embed(xf[:, 0], xf[:, 1], xf[:, 2], W0, W1, W2)
    return out.reshape(B, N, T, 3 * D)


# speedup vs baseline: 10.1371x; 10.1371x over previous
import functools

import jax
import jax.numpy as jnp
from jax import lax
from jax.experimental import pallas as pl
from jax.experimental.pallas import tpu as pltpu
from jax.experimental.pallas import tpu_sc as plsc

B, N, T = 1024, 26, 20
M = B * N * T
D = 16
NC, NS = 2, 16
NW = NC * NS
CHUNK = M // NW
SUB = 2080
N_ITERS = CHUNK // SUB

_mesh = plsc.VectorSubcoreMesh(core_axis_name="c", subcore_axis_name="s")


@functools.partial(
    pl.kernel,
    mesh=_mesh,
    compiler_params=pltpu.CompilerParams(use_tc_tiling_on_sc=False),
    out_type=jax.ShapeDtypeStruct((M, 3 * D), jnp.float32),
    scratch_types=[
        pltpu.VMEM((SUB,), jnp.int32),
        pltpu.VMEM((SUB, D), jnp.float32),
        pltpu.SemaphoreType.DMA,
    ],
)
def _embed(idx0, idx1, idx2, w0, w1, w2, out, idx_v, rows_v, sem):
    wid = lax.axis_index("s") * NC + lax.axis_index("c")
    base = wid * CHUNK
    idxs = (idx0, idx1, idx2)
    tables = (w0, w1, w2)

    def body(j, _):
        start = base + j * SUB
        for i in range(3):
            pltpu.sync_copy(idxs[i].at[pl.ds(start, SUB)], idx_v)
            pltpu.async_copy(tables[i].at[idx_v], rows_v, sem).wait()
            pltpu.sync_copy(rows_v, out.at[pl.ds(start, SUB), pl.ds(i * D, D)])
        return 0

    lax.fori_loop(0, N_ITERS, body, 0)


def kernel(x, W0, W1, W2):
    xf = x.reshape(M, 3)
    out = _embed(xf[:, 0], xf[:, 1], xf[:, 2], W0, W1, W2)
    return out.reshape(B, N, T, 3 * D)
